# Initial kernel scaffold; baseline (speedup 1.0000x reference)
import jax
import jax.numpy as jnp
from jax.experimental import pallas as pl


def kernel(x, edge_index, W_pre, b_pre, W0, b0, W1, b1, W_post, b_post):
    raise NotImplementedError("write your pallas kernel here")


# speedup vs baseline: 5.2005x; 5.2005x over previous
import functools

import jax
import jax.numpy as jnp
from jax import lax
from jax.experimental import pallas as pl
from jax.experimental.pallas import tpu as pltpu
from jax.experimental.pallas import tpu_sc as plsc

N_NODES = 10000
D = 128
E_EDGES = 320000

NC = 2
NS = 16
NW = NC * NS

EPT = E_EDGES // NW
CHUNK = 128
CH = -(-EPT // CHUNK)
CH_PAD = CH + 1
EPT_PAD = CH_PAD * CHUNK

ZROWS = 632
N_ACC = NS * ZROWS
OPT = ZROWS

ROW_BLK = 1000


def _sc_agg_body(h_hbm, edges_hbm, zeros_hbm, out_hbm,
                 idx_a, idx_b, rows_a, rows_b, agg_sh, sem_a, sem_b):
    c = lax.axis_index("c")
    s = lax.axis_index("s")
    w = c * NS + s

    pltpu.sync_copy(zeros_hbm, agg_sh.at[pl.ds(s * ZROWS, ZROWS)])
    pltpu.sync_copy(edges_hbm.at[w].at[0], idx_a)
    plsc.subcore_barrier()

    pltpu.async_copy(h_hbm.at[idx_a.at[0]], rows_a, sem_a)
    pltpu.sync_copy(edges_hbm.at[w].at[1], idx_b)

    def step(i, _):
        j = 2 * i
        pltpu.async_copy(h_hbm.at[idx_b.at[0]], rows_b, sem_b)
        pltpu.make_async_copy(h_hbm.at[idx_a.at[0]], rows_a, sem_a).wait()
        pltpu.sync_copy(rows_a, agg_sh.at[idx_a.at[1]], add=True)
        pltpu.sync_copy(edges_hbm.at[w].at[j + 2], idx_a)
        pltpu.async_copy(h_hbm.at[idx_a.at[0]], rows_a, sem_a)
        pltpu.make_async_copy(h_hbm.at[idx_b.at[0]], rows_b, sem_b).wait()
        pltpu.sync_copy(rows_b, agg_sh.at[idx_b.at[1]], add=True)
        pltpu.sync_copy(edges_hbm.at[w].at[j + 3], idx_b)
        return _

    lax.fori_loop(0, (CH - 1) // 2, step, None, unroll=False)
    pltpu.make_async_copy(h_hbm.at[idx_a.at[0]], rows_a, sem_a).wait()
    pltpu.sync_copy(rows_a, agg_sh.at[idx_a.at[1]], add=True)

    plsc.subcore_barrier()
    pltpu.sync_copy(agg_sh.at[pl.ds(s * OPT, OPT)],
                    out_hbm.at[c].at[pl.ds(s * OPT, OPT)])


_sc_agg = functools.partial(
    pl.kernel,
    out_type=jax.ShapeDtypeStruct((NC, N_ACC, D), jnp.float32),
    mesh=plsc.VectorSubcoreMesh(core_axis_name="c", subcore_axis_name="s"),
    scratch_types=[
        pltpu.VMEM((2, CHUNK), jnp.int32),
        pltpu.VMEM((2, CHUNK), jnp.int32),
        pltpu.VMEM((CHUNK, D), jnp.float32),
        pltpu.VMEM((CHUNK, D), jnp.float32),
        pltpu.VMEM_SHARED((N_ACC, D), jnp.float32),
        pltpu.SemaphoreType.DMA,
        pltpu.SemaphoreType.DMA,
    ],
)(_sc_agg_body)


def _pre_body(x_ref, w_ref, b_ref, o_ref):
    acc = jnp.dot(x_ref[...], w_ref[...], preferred_element_type=jnp.float32)
    o_ref[...] = jnp.maximum(acc + b_ref[...], 0.0)


def _mid_body(p_ref, w_ref, b_ref, o_ref):
    agg = p_ref[0] + p_ref[1]
    acc = jnp.dot(agg, w_ref[...], preferred_element_type=jnp.float32)
    o_ref[...] = jnp.maximum(acc + b_ref[...], 0.0)


def _final_body(p_ref, w1_ref, b1_ref, wp_ref, bp_ref, o_ref):
    agg = p_ref[0] + p_ref[1]
    h = jnp.dot(agg, w1_ref[...], preferred_element_type=jnp.float32)
    h = jnp.maximum(h + b1_ref[...], 0.0)
    nrm = jnp.sqrt(jnp.sum(h * h, axis=1, keepdims=True))
    h = h / (nrm + 1e-12)
    o_ref[...] = (jnp.dot(h, wp_ref[...], preferred_element_type=jnp.float32)
                  + bp_ref[...])


_W_SPEC = pl.BlockSpec((D, D), lambda i: (0, 0))
_B_SPEC = pl.BlockSpec((1, D), lambda i: (0, 0))
_X_SPEC = pl.BlockSpec((ROW_BLK, D), lambda i: (i, 0))
_P_SPEC = pl.BlockSpec((NC, ROW_BLK, D), lambda i: (0, i, 0))
_OUT_SHAPE = jax.ShapeDtypeStruct((N_NODES, D), jnp.float32)

_tc_pre = pl.pallas_call(
    _pre_body, grid=(N_NODES // ROW_BLK,),
    in_specs=[_X_SPEC, _W_SPEC, _B_SPEC],
    out_specs=_X_SPEC, out_shape=_OUT_SHAPE)

_tc_mid = pl.pallas_call(
    _mid_body, grid=(N_NODES // ROW_BLK,),
    in_specs=[_P_SPEC, _W_SPEC, _B_SPEC],
    out_specs=_X_SPEC, out_shape=_OUT_SHAPE)

_tc_final = pl.pallas_call(
    _final_body, grid=(N_NODES // ROW_BLK,),
    in_specs=[_P_SPEC, _W_SPEC, _B_SPEC, _W_SPEC, _B_SPEC],
    out_specs=_X_SPEC, out_shape=_OUT_SHAPE)


def kernel(x, edge_index, W_pre, b_pre, W0, b0, W1, b1, W_post, b_post):
    src = edge_index[0]
    dst = edge_index[1]
    pad = NW * CH * CHUNK - E_EDGES
    src3 = jnp.concatenate(
        [src, jnp.zeros((pad,), jnp.int32)]).reshape(NW, CH, CHUNK)
    dst3 = jnp.concatenate(
        [dst, jnp.full((pad,), N_NODES, jnp.int32)]).reshape(NW, CH, CHUNK)
    edges = jnp.stack([src3, dst3], axis=2)
    edges = jnp.pad(edges, ((0, 0), (0, 1), (0, 0), (0, 0)))
    zeros = jnp.zeros((ZROWS, D), jnp.float32)

    b_pre2 = b_pre.reshape(1, D)
    b02 = b0.reshape(1, D)
    b12 = b1.reshape(1, D)
    b_post2 = b_post.reshape(1, D)

    h = _tc_pre(x, W_pre, b_pre2)
    p = _sc_agg(h, edges, zeros)
    h = _tc_mid(p, W0, b02)
    p = _sc_agg(h, edges, zeros)
    return _tc_final(p, W1, b12, W_post, b_post2)
